# baseline (speedup 1.0000x reference)
import jax
import jax.numpy as jnp
from jax import lax
from jax.experimental import pallas as pl
from jax.experimental.pallas import tpu as pltpu

VOCAB = 100000
DIM = 300
MAX_NORM = 1.0
BATCH = 1024

_RPB = 64
_NB = BATCH // _RPB

_TV = 1024
_NV = (VOCAB + _TV - 1) // _TV


def _gather_body(idx_ref, *refs):
    out_ref = refs[-1]
    for k in range(_RPB):
        row = refs[k][0]
        ss = jnp.sum(row * row, axis=1, keepdims=True)
        scale = jnp.minimum(1.0, MAX_NORM * lax.rsqrt(jnp.maximum(ss, 1e-14)))
        out_ref[k : k + 1, :] = (row * scale).astype(jnp.bfloat16)


def _row_spec(k):
    return pl.BlockSpec(
        (1, 1, DIM), lambda i, idx_ref, k=k: (idx_ref[i * _RPB + k], 0, 0)
    )


@jax.jit
def _gather_clip(idx, emb):
    grid_spec = pltpu.PrefetchScalarGridSpec(
        num_scalar_prefetch=1,
        grid=(_NB,),
        in_specs=[_row_spec(k) for k in range(_RPB)],
        out_specs=pl.BlockSpec((_RPB, DIM), lambda i, idx_ref: (i, 0)),
    )
    emb3 = emb.reshape(VOCAB, 1, DIM)
    return pl.pallas_call(
        _gather_body,
        grid_spec=grid_spec,
        out_shape=jax.ShapeDtypeStruct((BATCH, DIM), jnp.bfloat16),
    )(idx, *([emb3] * _RPB))


def _mm_body(x_ref, w_ref, b_ref, o_ref):
    xb = x_ref[...]
    wb = w_ref[...].astype(jnp.bfloat16)
    acc = lax.dot_general(
        xb, wb, (((1,), (1,)), ((), ())), preferred_element_type=jnp.float32
    )
    o_ref[...] = acc + b_ref[...]


@jax.jit
def _tc_project(x, W, b2d):
    return pl.pallas_call(
        _mm_body,
        grid=(_NV,),
        in_specs=[
            pl.BlockSpec((BATCH, DIM), lambda i: (0, 0)),
            pl.BlockSpec((_TV, DIM), lambda i: (i, 0)),
            pl.BlockSpec((1, _TV), lambda i: (0, i)),
        ],
        out_specs=pl.BlockSpec((BATCH, _TV), lambda i: (0, i)),
        out_shape=jax.ShapeDtypeStruct((BATCH, VOCAB), jnp.float32),
    )(x, W, b2d)


def kernel(inputs_, emb, W, b):
    idx = inputs_.astype(jnp.int32)
    x = _gather_clip(idx, emb)
    return _tc_project(x, W, b.reshape(1, VOCAB))


# speedup vs baseline: 1.2578x; 1.2578x over previous
import jax
import jax.numpy as jnp
from jax import lax
from jax.experimental import pallas as pl
from jax.experimental.pallas import tpu as pltpu

VOCAB = 100000
DIM = 300
MAX_NORM = 1.0
BATCH = 1024

_RPB = 64
_NB = BATCH // _RPB

_TV = 2048
_NV = (VOCAB + _TV - 1) // _TV


def _gather_body(idx_ref, *refs):
    out_ref = refs[-1]
    i = pl.program_id(0)
    for k in range(_RPB):
        blk = refs[k][...]
        r = idx_ref[i * _RPB + k] % 8
        sel = lax.broadcasted_iota(jnp.int32, (8, 1), 0) == r
        row = jnp.sum(jnp.where(sel, blk, 0.0), axis=0, keepdims=True)
        ss = jnp.sum(row * row, axis=1, keepdims=True)
        scale = jnp.minimum(1.0, MAX_NORM * lax.rsqrt(jnp.maximum(ss, 1e-14)))
        out_ref[k : k + 1, :] = (row * scale).astype(jnp.bfloat16)


def _row_spec(k):
    return pl.BlockSpec(
        (8, DIM), lambda i, idx_ref, k=k: (idx_ref[i * _RPB + k] // 8, 0)
    )


@jax.jit
def _gather_clip(idx, emb):
    grid_spec = pltpu.PrefetchScalarGridSpec(
        num_scalar_prefetch=1,
        grid=(_NB,),
        in_specs=[_row_spec(k) for k in range(_RPB)],
        out_specs=pl.BlockSpec((_RPB, DIM), lambda i, idx_ref: (i, 0)),
    )
    return pl.pallas_call(
        _gather_body,
        grid_spec=grid_spec,
        out_shape=jax.ShapeDtypeStruct((BATCH, DIM), jnp.bfloat16),
    )(idx, *([emb] * _RPB))


def _mm_body(x_ref, w_ref, b_ref, o_ref):
    xb = x_ref[...]
    wb = w_ref[...].astype(jnp.bfloat16)
    acc = lax.dot_general(
        xb, wb, (((1,), (1,)), ((), ())), preferred_element_type=jnp.float32
    )
    o_ref[...] = acc + b_ref[...]


@jax.jit
def _tc_project(x, W, b2d):
    return pl.pallas_call(
        _mm_body,
        grid=(_NV,),
        in_specs=[
            pl.BlockSpec((BATCH, DIM), lambda i: (0, 0)),
            pl.BlockSpec((_TV, DIM), lambda i: (i, 0)),
            pl.BlockSpec((1, _TV), lambda i: (0, i)),
        ],
        out_specs=pl.BlockSpec((BATCH, _TV), lambda i: (0, i)),
        out_shape=jax.ShapeDtypeStruct((BATCH, VOCAB), jnp.float32),
    )(x, W, b2d)


def kernel(inputs_, emb, W, b):
    idx = inputs_.astype(jnp.int32)
    x = _gather_clip(idx, emb)
    return _tc_project(x, W, b.reshape(1, VOCAB))


# speedup vs baseline: 1.2952x; 1.0297x over previous
import jax
import jax.numpy as jnp
from jax import lax
from jax.experimental import pallas as pl
from jax.experimental.pallas import tpu as pltpu

VOCAB = 100000
DIM = 300
MAX_NORM = 1.0
BATCH = 1024

_RPB = 64
_NB = BATCH // _RPB

_TV = 2048
_NV = (VOCAB + _TV - 1) // _TV


def _gather_body(idx_ref, emb_ref, out_ref, xs_ref, sem):
    def issue(j, _):
        cp = pltpu.make_async_copy(
            emb_ref.at[pl.ds(idx_ref[j], 1)], xs_ref.at[pl.ds(j, 1)], sem
        )
        cp.start()
        return 0

    lax.fori_loop(0, BATCH, issue, 0)

    def drain(j, _):
        pltpu.make_async_copy(
            emb_ref.at[pl.ds(0, 1)], xs_ref.at[pl.ds(0, 1)], sem
        ).wait()
        return 0

    lax.fori_loop(0, BATCH, drain, 0)
    x = xs_ref[...]
    ss = jnp.sum(x * x, axis=1, keepdims=True)
    scale = jnp.minimum(1.0, MAX_NORM * lax.rsqrt(jnp.maximum(ss, 1e-14)))
    out_ref[...] = (x * scale).astype(jnp.bfloat16)


@jax.jit
def _gather_clip(idx, emb):
    grid_spec = pltpu.PrefetchScalarGridSpec(
        num_scalar_prefetch=1,
        grid=(1,),
        in_specs=[pl.BlockSpec(memory_space=pltpu.MemorySpace.HBM)],
        out_specs=pl.BlockSpec((BATCH, DIM), lambda i, idx_ref: (0, 0)),
        scratch_shapes=[
            pltpu.VMEM((BATCH, DIM), jnp.float32),
            pltpu.SemaphoreType.DMA,
        ],
    )
    return pl.pallas_call(
        _gather_body,
        grid_spec=grid_spec,
        out_shape=jax.ShapeDtypeStruct((BATCH, DIM), jnp.bfloat16),
    )(idx, emb)


def _mm_body(x_ref, w_ref, b_ref, o_ref):
    xb = x_ref[...]
    wb = w_ref[...].astype(jnp.bfloat16)
    acc = lax.dot_general(
        xb, wb, (((1,), (1,)), ((), ())), preferred_element_type=jnp.float32
    )
    o_ref[...] = acc + b_ref[...]


@jax.jit
def _tc_project(x, W, b2d):
    return pl.pallas_call(
        _mm_body,
        grid=(_NV,),
        in_specs=[
            pl.BlockSpec((BATCH, DIM), lambda i: (0, 0)),
            pl.BlockSpec((_TV, DIM), lambda i: (i, 0)),
            pl.BlockSpec((1, _TV), lambda i: (0, i)),
        ],
        out_specs=pl.BlockSpec((BATCH, _TV), lambda i: (0, i)),
        out_shape=jax.ShapeDtypeStruct((BATCH, VOCAB), jnp.float32),
    )(x, W, b2d)


def kernel(inputs_, emb, W, b):
    idx = inputs_.astype(jnp.int32)
    x = _gather_clip(idx, emb)
    return _tc_project(x, W, b.reshape(1, VOCAB))
